# Initial kernel scaffold; baseline (speedup 1.0000x reference)
import jax
import jax.numpy as jnp
from jax.experimental import pallas as pl


def kernel(msg, emb_weight):
    raise NotImplementedError("write your pallas kernel here")


# speedup vs baseline: 559.3473x; 559.3473x over previous
import functools

import jax
import jax.numpy as jnp
from jax.experimental import pallas as pl
from jax.experimental.pallas import tpu as pltpu

N_BITS_CONST = 100
MODEL_DIM_CONST = 64
ROWS_PER_BLOCK = 2048


def _tc_body(msg_ref, w_ref, out_ref):
    w = w_ref[...]
    diff = w[:, 1, :] - w[:, 0, :]
    base = jnp.sum(w[:, 0, :], axis=0)
    m = msg_ref[...].astype(jnp.float32)
    acc = jax.lax.dot_general(
        m, diff, (((1,), (0,)), ((), ())),
        preferred_element_type=jnp.float32)
    out_ref[...] = acc + base[None, :]


def kernel(msg, emb_weight):
    n_batch, n_bits = msg.shape
    two_n, model_dim = emb_weight.shape
    w3 = emb_weight.reshape(n_bits, 2, model_dim)
    grid = n_batch // ROWS_PER_BLOCK
    return pl.pallas_call(
        _tc_body,
        grid=(grid,),
        in_specs=[
            pl.BlockSpec((ROWS_PER_BLOCK, n_bits), lambda i: (i, 0)),
            pl.BlockSpec((n_bits, 2, model_dim), lambda i: (0, 0, 0)),
        ],
        out_specs=pl.BlockSpec((ROWS_PER_BLOCK, model_dim), lambda i: (i, 0)),
        out_shape=jax.ShapeDtypeStruct((n_batch, model_dim), jnp.float32),
    )(msg, w3)
